# Initial kernel scaffold; baseline (speedup 1.0000x reference)
import jax
import jax.numpy as jnp
from jax.experimental import pallas as pl


def kernel(feat, edge_index, W_src, b_src, W_dst, b_dst, W_val, b_val, attn):
    raise NotImplementedError("write your pallas kernel here")


# speedup vs baseline: 1.0468x; 1.0468x over previous
import jax
import jax.numpy as jnp
from jax.experimental import pallas as pl
from jax.experimental.pallas import tpu as pltpu

N = 10000
E = 320000
IN_FEATS = 128
HEADS = 4
OUT_FEATS = 32
HD = HEADS * OUT_FEATS
NEG_SLOPE = 0.2


def _proj_body(feat_ref, w_ref, b_ref, out_ref):
    out_ref[...] = (
        jnp.dot(feat_ref[...], w_ref[...], preferred_element_type=jnp.float32)
        + b_ref[...]
    )


def _project(feat, w_all_t, b_all):
    blk = 1000
    return pl.pallas_call(
        _proj_body,
        grid=(N // blk,),
        in_specs=[
            pl.BlockSpec((blk, IN_FEATS), lambda i: (i, 0)),
            pl.BlockSpec((IN_FEATS, 3 * HD), lambda i: (0, 0)),
            pl.BlockSpec((1, 3 * HD), lambda i: (0, 0)),
        ],
        out_specs=pl.BlockSpec((blk, 3 * HD), lambda i: (i, 0)),
        out_shape=jax.ShapeDtypeStruct((N, 3 * HD), jnp.float32),
    )(feat, w_all_t, b_all)


def kernel(feat, edge_index, W_src, b_src, W_dst, b_dst, W_val, b_val, attn):
    src = edge_index[0]
    dst = edge_index[1]
    w_all_t = jnp.concatenate([W_src, W_dst, W_val], axis=0).T
    b_all = jnp.concatenate([b_src, b_dst, b_val])[None, :]
    proj = _project(feat, w_all_t, b_all)
    el = proj[:, :HD].reshape(N, HEADS, OUT_FEATS)
    er = proj[:, HD:2 * HD].reshape(N, HEADS, OUT_FEATS)
    v = proj[:, 2 * HD:].reshape(N, HEADS, OUT_FEATS)
    e = el[src] + er[dst]
    e = jnp.where(e > 0, e, NEG_SLOPE * e)
    logits = (e * attn).sum(axis=-1)
    ex = jnp.exp(logits)
    seg_sum = jax.ops.segment_sum(ex, dst, num_segments=N)
    a = ex / (seg_sum[dst] + 1e-9)
    msg = v[src] * a[:, :, None]
    return jax.ops.segment_sum(msg, dst, num_segments=N)


# speedup vs baseline: 8.5056x; 8.1253x over previous
import functools

import jax
import jax.numpy as jnp
from jax import lax
from jax.experimental import pallas as pl
from jax.experimental.pallas import tpu as pltpu
from jax.experimental.pallas import tpu_sc as plsc

N = 10000
E = 320000
IN_FEATS = 128
HEADS = 4
OUT_FEATS = 32
HD = HEADS * OUT_FEATS
NEG_SLOPE = 0.2

NW = 32
EPW = E // NW
CA = 400
NCHUNK = EPW // CA
NGROUP = CA // 16
CD = 80
NCHUNK_D = EPW // CD
NGROUP_D = CD // 16


def _proj_body(feat_ref, w_ref, b_ref, el_ref, er_ref, v_ref):
    r = (
        jnp.dot(feat_ref[...], w_ref[...], preferred_element_type=jnp.float32)
        + b_ref[...]
    )
    el_ref[...] = r[:, :HD]
    er_ref[...] = r[:, HD:2 * HD]
    v_ref[...] = r[:, 2 * HD:]


def _project(feat, w_all_t, b_all):
    blk = 1000
    return pl.pallas_call(
        _proj_body,
        grid=(N // blk,),
        in_specs=[
            pl.BlockSpec((blk, IN_FEATS), lambda i: (i, 0)),
            pl.BlockSpec((IN_FEATS, 3 * HD), lambda i: (0, 0)),
            pl.BlockSpec((1, 3 * HD), lambda i: (0, 0)),
        ],
        out_specs=[
            pl.BlockSpec((blk, HD), lambda i: (i, 0)),
            pl.BlockSpec((blk, HD), lambda i: (i, 0)),
            pl.BlockSpec((blk, HD), lambda i: (i, 0)),
        ],
        out_shape=[
            jax.ShapeDtypeStruct((N, HD), jnp.float32),
            jax.ShapeDtypeStruct((N, HD), jnp.float32),
            jax.ShapeDtypeStruct((N, HD), jnp.float32),
        ],
    )(feat, w_all_t, b_all)


def _sc_pass_a(el, er, src, dst, attn_b, z4):
    mesh = plsc.VectorSubcoreMesh(core_axis_name="c", subcore_axis_name="s")

    @functools.partial(
        pl.kernel,
        mesh=mesh,
        compiler_params=pltpu.CompilerParams(needs_layout_passes=False),
        out_type=[
            jax.ShapeDtypeStruct((E * 4,), jnp.float32),
            jax.ShapeDtypeStruct((2, N * 4), jnp.float32),
        ],
        scratch_types=[
            pltpu.VMEM((CA,), jnp.int32),
            pltpu.VMEM((CA,), jnp.int32),
            pltpu.VMEM((CA * 4,), jnp.int32),
            pltpu.VMEM((CA, HD), jnp.float32),
            pltpu.VMEM((CA, HD), jnp.float32),
            pltpu.VMEM((CA * 4,), jnp.float32),
            pltpu.VMEM((HD, 16), jnp.float32),
            pltpu.VMEM_SHARED((N * 4,), jnp.float32),
            pltpu.SemaphoreType.DMA,
            pltpu.SemaphoreType.DMA,
        ],
    )
    def k(el_h, er_h, src_h, dst_h, attnb_h, z4_h, ex_h, segp_h,
          idx_s, idx_d, idx4, rows_el, rows_er, exflat, attn_v, seg_sh,
          sem1, sem2):
        c = lax.axis_index("c")
        s = lax.axis_index("s")
        w = s * 2 + c
        iota = lax.iota(jnp.int32, 16)
        zeros16 = jnp.zeros((16,), jnp.float32)
        pltpu.sync_copy(attnb_h, attn_v)

        @pl.when(s == 0)
        def _():
            pltpu.sync_copy(z4_h, seg_sh)

        plsc.subcore_barrier()

        def chunk(kk, carry):
            base = w * EPW + kk * CA
            pltpu.sync_copy(src_h.at[pl.ds(base, CA)], idx_s)
            pltpu.sync_copy(dst_h.at[pl.ds(base, CA)], idx_d)
            cp1 = pltpu.async_copy(el_h.at[idx_s], rows_el, sem1)
            cp2 = pltpu.async_copy(er_h.at[idx_d], rows_er, sem2)

            def mkidx(kk2, carry2):
                si = kk2 * 16 + iota
                dstv = plsc.load_gather(idx_d, [si >> 2])
                idx4[pl.ds(kk2 * 16, 16)] = dstv * 4 + (si & 3)
                return carry2

            lax.fori_loop(0, CA * 4 // 16, mkidx, None)
            cp1.wait()
            cp2.wait()

            def group(g, gcarry):
                ri = g * 16 + iota
                acc = [zeros16, zeros16, zeros16, zeros16]
                for d in range(HD):
                    cd = jnp.full((16,), d, jnp.int32)
                    ve = plsc.load_gather(rows_el, [ri, cd])
                    vr = plsc.load_gather(rows_er, [ri, cd])
                    x = ve + vr
                    x = jnp.maximum(x, x * NEG_SLOPE)
                    acc[d // OUT_FEATS] = acc[d // OUT_FEATS] + x * attn_v[d, :]
                for h in range(HEADS):
                    plsc.store_scatter(exflat, [ri * 4 + h], jnp.exp(acc[h]))
                return gcarry

            lax.fori_loop(0, NGROUP, group, None)
            pltpu.sync_copy(exflat, ex_h.at[pl.ds(base * 4, CA * 4)])
            pltpu.sync_copy(exflat, seg_sh.at[idx4], add=True)
            return carry

        lax.fori_loop(0, NCHUNK, chunk, None)
        plsc.subcore_barrier()

        @pl.when(s == 0)
        def _():
            pltpu.sync_copy(seg_sh, segp_h.at[c])

    return k(el, er, src, dst, attn_b, z4)


def _recip_body(p_ref, o_ref):
    o_ref[...] = 1.0 / (p_ref[0] + p_ref[1] + 1e-9)


def _recip(segp):
    return pl.pallas_call(
        _recip_body,
        out_shape=jax.ShapeDtypeStruct((N * 4,), jnp.float32),
    )(segp)


def _sc_pass_d(v, src, dst, ex, r, z128):
    mesh = plsc.VectorSubcoreMesh(core_axis_name="c", subcore_axis_name="s")

    @functools.partial(
        pl.kernel,
        mesh=mesh,
        compiler_params=pltpu.CompilerParams(needs_layout_passes=False),
        out_type=jax.ShapeDtypeStruct((2, N, HD), jnp.float32),
        scratch_types=[
            pltpu.VMEM((CD,), jnp.int32),
            pltpu.VMEM((CD,), jnp.int32),
            pltpu.VMEM((CD * 4,), jnp.int32),
            pltpu.VMEM((CD, HD), jnp.float32),
            pltpu.VMEM((CD, HD), jnp.float32),
            pltpu.VMEM((CD * 4,), jnp.float32),
            pltpu.VMEM((CD * 4,), jnp.float32),
            pltpu.VMEM_SHARED((N, HD), jnp.float32),
            pltpu.SemaphoreType.DMA,
            pltpu.SemaphoreType.DMA,
        ],
    )
    def k(v_h, src_h, dst_h, ex_h, r_h, z128_h, rst_h,
          idx_s, idx_d, idx4, v_rows, msg_rows, exflat, rflat, rst_sh,
          sem1, sem2):
        c = lax.axis_index("c")
        s = lax.axis_index("s")
        w = s * 2 + c
        iota = lax.iota(jnp.int32, 16)

        @pl.when(s == 0)
        def _():
            pltpu.sync_copy(z128_h, rst_sh)

        plsc.subcore_barrier()

        def chunk(kk, carry):
            base = w * EPW + kk * CD
            pltpu.sync_copy(src_h.at[pl.ds(base, CD)], idx_s)
            pltpu.sync_copy(dst_h.at[pl.ds(base, CD)], idx_d)
            cp1 = pltpu.async_copy(v_h.at[idx_s], v_rows, sem1)

            def mkidx(kk2, carry2):
                si = kk2 * 16 + iota
                dstv = plsc.load_gather(idx_d, [si >> 2])
                idx4[pl.ds(kk2 * 16, 16)] = dstv * 4 + (si & 3)
                return carry2

            lax.fori_loop(0, CD * 4 // 16, mkidx, None)
            cp2 = pltpu.async_copy(r_h.at[idx4], rflat, sem2)
            pltpu.sync_copy(ex_h.at[pl.ds(base * 4, CD * 4)], exflat)
            cp1.wait()
            cp2.wait()

            def amul(kk2, carry2):
                sl = pl.ds(kk2 * 16, 16)
                exflat[sl] = exflat[sl] * rflat[sl]
                return carry2

            lax.fori_loop(0, CD * 4 // 16, amul, None)

            def group(g, gcarry):
                ri = g * 16 + iota
                a = [plsc.load_gather(exflat, [ri * 4 + h])
                     for h in range(HEADS)]
                for d in range(HD):
                    cd = jnp.full((16,), d, jnp.int32)
                    vv = plsc.load_gather(v_rows, [ri, cd])
                    plsc.store_scatter(msg_rows, [ri, cd],
                                       vv * a[d // OUT_FEATS])
                return gcarry

            lax.fori_loop(0, NGROUP_D, group, None)
            pltpu.sync_copy(msg_rows, rst_sh.at[idx_d], add=True)
            return carry

        lax.fori_loop(0, NCHUNK_D, chunk, None)
        plsc.subcore_barrier()

        @pl.when(s == 0)
        def _():
            pltpu.sync_copy(rst_sh, rst_h.at[c])

    return k(v, src, dst, ex, r, z128)


def _combine_body(p_ref, o_ref):
    o_ref[...] = p_ref[0] + p_ref[1]


def _combine(rstp):
    blk = 1000
    return pl.pallas_call(
        _combine_body,
        grid=(N // blk,),
        in_specs=[pl.BlockSpec((2, blk, HD), lambda i: (0, i, 0))],
        out_specs=pl.BlockSpec((blk, HD), lambda i: (i, 0)),
        out_shape=jax.ShapeDtypeStruct((N, HD), jnp.float32),
    )(rstp)


def kernel(feat, edge_index, W_src, b_src, W_dst, b_dst, W_val, b_val, attn):
    src = edge_index[0]
    dst = edge_index[1]
    w_all_t = jnp.concatenate([W_src, W_dst, W_val], axis=0).T
    b_all = jnp.concatenate([b_src, b_dst, b_val])[None, :]
    el, er, v = _project(feat, w_all_t, b_all)
    attn_b = jnp.broadcast_to(attn.reshape(HD, 1), (HD, 16))
    z4 = jnp.zeros((N * 4,), jnp.float32)
    z128 = jnp.zeros((N, HD), jnp.float32)
    ex, segp = _sc_pass_a(el, er, src, dst, attn_b, z4)
    r = _recip(segp)
    rstp = _sc_pass_d(v, src, dst, ex, r, z128)
    rst = _combine(rstp)
    return rst.reshape(N, HEADS, OUT_FEATS)


# speedup vs baseline: 8.8956x; 1.0459x over previous
import functools

import jax
import jax.numpy as jnp
from jax import lax
from jax.experimental import pallas as pl
from jax.experimental.pallas import tpu as pltpu
from jax.experimental.pallas import tpu_sc as plsc

N = 10000
E = 320000
IN_FEATS = 128
HEADS = 4
OUT_FEATS = 32
HD = HEADS * OUT_FEATS
NEG_SLOPE = 0.2

NW = 32
EPW = E // NW
CC = 80
NCH = EPW // CC
NGR = CC // 16
NPAIR = (NCH - 1) // 2


def _proj_body(feat_ref, w_ref, b_ref, el_ref, er_ref, v_ref):
    r = (
        jnp.dot(feat_ref[...], w_ref[...], preferred_element_type=jnp.float32)
        + b_ref[...]
    )
    el_ref[...] = r[:, :HD]
    er_ref[...] = r[:, HD:2 * HD]
    v_ref[...] = r[:, 2 * HD:]


def _project(feat, w_all_t, b_all):
    blk = 1000
    return pl.pallas_call(
        _proj_body,
        grid=(N // blk,),
        in_specs=[
            pl.BlockSpec((blk, IN_FEATS), lambda i: (i, 0)),
            pl.BlockSpec((IN_FEATS, 3 * HD), lambda i: (0, 0)),
            pl.BlockSpec((1, 3 * HD), lambda i: (0, 0)),
        ],
        out_specs=[
            pl.BlockSpec((blk, HD), lambda i: (i, 0)),
            pl.BlockSpec((blk, HD), lambda i: (i, 0)),
            pl.BlockSpec((blk, HD), lambda i: (i, 0)),
        ],
        out_shape=[
            jax.ShapeDtypeStruct((N, HD), jnp.float32),
            jax.ShapeDtypeStruct((N, HD), jnp.float32),
            jax.ShapeDtypeStruct((N, HD), jnp.float32),
        ],
    )(feat, w_all_t, b_all)


def _sc_pass_a(el, er, src, dst, attn_b, z4):
    mesh = plsc.VectorSubcoreMesh(core_axis_name="c", subcore_axis_name="s")

    @functools.partial(
        pl.kernel,
        mesh=mesh,
        compiler_params=pltpu.CompilerParams(needs_layout_passes=False),
        out_type=[
            jax.ShapeDtypeStruct((E * 4,), jnp.float32),
            jax.ShapeDtypeStruct((2, N * 4), jnp.float32),
        ],
        scratch_types=[
            pltpu.VMEM((CC,), jnp.int32),
            pltpu.VMEM((CC,), jnp.int32),
            pltpu.VMEM((CC * 4,), jnp.int32),
            pltpu.VMEM((CC, HD), jnp.float32),
            pltpu.VMEM((CC, HD), jnp.float32),
            pltpu.VMEM((CC,), jnp.int32),
            pltpu.VMEM((CC,), jnp.int32),
            pltpu.VMEM((CC * 4,), jnp.int32),
            pltpu.VMEM((CC, HD), jnp.float32),
            pltpu.VMEM((CC, HD), jnp.float32),
            pltpu.VMEM((CC * 4,), jnp.float32),
            pltpu.VMEM((HD, 16), jnp.float32),
            pltpu.VMEM_SHARED((N * 4,), jnp.float32),
            pltpu.SemaphoreType.DMA,
            pltpu.SemaphoreType.DMA,
        ],
    )
    def k(el_h, er_h, src_h, dst_h, attnb_h, z4_h, ex_h, segp_h,
          ids0, idd0, idx40, rel0, rer0,
          ids1, idd1, idx41, rel1, rer1,
          exflat, attn_v, seg_sh, sem0, sem1):
        c = lax.axis_index("c")
        s = lax.axis_index("s")
        w = s * 2 + c
        iota = lax.iota(jnp.int32, 16)
        zeros16 = jnp.zeros((16,), jnp.float32)
        pltpu.sync_copy(attnb_h, attn_v)

        @pl.when(s == 0)
        def _():
            pltpu.sync_copy(z4_h, seg_sh)

        plsc.subcore_barrier()

        bufs = ((ids0, idd0, idx40, rel0, rer0, sem0),
                (ids1, idd1, idx41, rel1, rer1, sem1))

        def issue(kk, b):
            ids, idd, idx4, rel, rer, sem = bufs[b]
            base = w * EPW + kk * CC
            pltpu.sync_copy(src_h.at[pl.ds(base, CC)], ids)
            pltpu.sync_copy(dst_h.at[pl.ds(base, CC)], idd)

            def mkidx(kk2, carry2):
                si = kk2 * 16 + iota
                dstv = plsc.load_gather(idd, [si >> 2])
                idx4[pl.ds(kk2 * 16, 16)] = dstv * 4 + (si & 3)
                return carry2

            lax.fori_loop(0, CC * 4 // 16, mkidx, None)
            pltpu.async_copy(el_h.at[ids], rel, sem)
            pltpu.async_copy(er_h.at[idd], rer, sem)

        def stage(kk, b):
            ids, idd, idx4, rel, rer, sem = bufs[b]
            pltpu.make_async_copy(el_h.at[ids], rel, sem).wait()
            pltpu.make_async_copy(er_h.at[idd], rer, sem).wait()

            def group(g, gcarry):
                ri = g * 16 + iota
                acc = [zeros16, zeros16, zeros16, zeros16]
                for d in range(HD):
                    cd = jnp.full((16,), d, jnp.int32)
                    ve = plsc.load_gather(rel, [ri, cd])
                    vr = plsc.load_gather(rer, [ri, cd])
                    x = ve + vr
                    x = jnp.maximum(x, x * NEG_SLOPE)
                    acc[d // OUT_FEATS] = acc[d // OUT_FEATS] + x * attn_v[d, :]
                for h in range(HEADS):
                    plsc.store_scatter(exflat, [ri * 4 + h], jnp.exp(acc[h]))
                return gcarry

            lax.fori_loop(0, NGR, group, None)
            base = w * EPW + kk * CC
            pltpu.sync_copy(exflat, ex_h.at[pl.ds(base * 4, CC * 4)])
            pltpu.sync_copy(exflat, seg_sh.at[idx4], add=True)

        issue(0, 0)

        def pair(jj, carry):
            issue(2 * jj + 1, 1)
            stage(2 * jj, 0)
            issue(2 * jj + 2, 0)
            stage(2 * jj + 1, 1)
            return carry

        lax.fori_loop(0, NPAIR, pair, None)
        stage(NCH - 1, 0)
        plsc.subcore_barrier()

        @pl.when(s == 0)
        def _():
            pltpu.sync_copy(seg_sh, segp_h.at[c])

    return k(el, er, src, dst, attn_b, z4)


def _recip_body(p_ref, o_ref):
    o_ref[...] = 1.0 / (p_ref[0] + p_ref[1] + 1e-9)


def _recip(segp):
    return pl.pallas_call(
        _recip_body,
        out_shape=jax.ShapeDtypeStruct((N * 4,), jnp.float32),
    )(segp)


def _sc_pass_d(v, src, dst, ex, r, z128):
    mesh = plsc.VectorSubcoreMesh(core_axis_name="c", subcore_axis_name="s")

    @functools.partial(
        pl.kernel,
        mesh=mesh,
        compiler_params=pltpu.CompilerParams(needs_layout_passes=False),
        out_type=jax.ShapeDtypeStruct((2, N, HD), jnp.float32),
        scratch_types=[
            pltpu.VMEM((CC,), jnp.int32),
            pltpu.VMEM((CC,), jnp.int32),
            pltpu.VMEM((CC * 4,), jnp.int32),
            pltpu.VMEM((CC, HD), jnp.float32),
            pltpu.VMEM((CC * 4,), jnp.float32),
            pltpu.VMEM((CC * 4,), jnp.float32),
            pltpu.VMEM((CC,), jnp.int32),
            pltpu.VMEM((CC,), jnp.int32),
            pltpu.VMEM((CC * 4,), jnp.int32),
            pltpu.VMEM((CC, HD), jnp.float32),
            pltpu.VMEM((CC * 4,), jnp.float32),
            pltpu.VMEM((CC * 4,), jnp.float32),
            pltpu.VMEM((CC, HD), jnp.float32),
            pltpu.VMEM_SHARED((N, HD), jnp.float32),
            pltpu.SemaphoreType.DMA,
            pltpu.SemaphoreType.DMA,
        ],
    )
    def k(v_h, src_h, dst_h, ex_h, r_h, z128_h, rst_h,
          ids0, idd0, idx40, vr0, exf0, rf0,
          ids1, idd1, idx41, vr1, exf1, rf1,
          msg_rows, rst_sh, sem0, sem1):
        c = lax.axis_index("c")
        s = lax.axis_index("s")
        w = s * 2 + c
        iota = lax.iota(jnp.int32, 16)

        @pl.when(s == 0)
        def _():
            pltpu.sync_copy(z128_h, rst_sh)

        plsc.subcore_barrier()

        bufs = ((ids0, idd0, idx40, vr0, exf0, rf0, sem0),
                (ids1, idd1, idx41, vr1, exf1, rf1, sem1))

        def issue(kk, b):
            ids, idd, idx4, vr, exf, rf, sem = bufs[b]
            base = w * EPW + kk * CC
            pltpu.sync_copy(src_h.at[pl.ds(base, CC)], ids)
            pltpu.sync_copy(dst_h.at[pl.ds(base, CC)], idd)

            def mkidx(kk2, carry2):
                si = kk2 * 16 + iota
                dstv = plsc.load_gather(idd, [si >> 2])
                idx4[pl.ds(kk2 * 16, 16)] = dstv * 4 + (si & 3)
                return carry2

            lax.fori_loop(0, CC * 4 // 16, mkidx, None)
            pltpu.async_copy(v_h.at[ids], vr, sem)
            pltpu.async_copy(r_h.at[idx4], rf, sem)
            pltpu.async_copy(ex_h.at[pl.ds(base * 4, CC * 4)], exf, sem)

        def stage(kk, b):
            ids, idd, idx4, vr, exf, rf, sem = bufs[b]
            base = w * EPW + kk * CC
            pltpu.make_async_copy(v_h.at[ids], vr, sem).wait()
            pltpu.make_async_copy(r_h.at[idx4], rf, sem).wait()
            pltpu.make_async_copy(ex_h.at[pl.ds(base * 4, CC * 4)], exf,
                                  sem).wait()

            def amul(kk2, carry2):
                sl = pl.ds(kk2 * 16, 16)
                exf[sl] = exf[sl] * rf[sl]
                return carry2

            lax.fori_loop(0, CC * 4 // 16, amul, None)

            def group(g, gcarry):
                ri = g * 16 + iota
                a = [plsc.load_gather(exf, [ri * 4 + h])
                     for h in range(HEADS)]
                for d in range(HD):
                    cd = jnp.full((16,), d, jnp.int32)
                    vv = plsc.load_gather(vr, [ri, cd])
                    plsc.store_scatter(msg_rows, [ri, cd],
                                       vv * a[d // OUT_FEATS])
                return gcarry

            lax.fori_loop(0, NGR, group, None)
            pltpu.sync_copy(msg_rows, rst_sh.at[idd], add=True)

        issue(0, 0)

        def pair(jj, carry):
            issue(2 * jj + 1, 1)
            stage(2 * jj, 0)
            issue(2 * jj + 2, 0)
            stage(2 * jj + 1, 1)
            return carry

        lax.fori_loop(0, NPAIR, pair, None)
        stage(NCH - 1, 0)
        plsc.subcore_barrier()

        @pl.when(s == 0)
        def _():
            pltpu.sync_copy(rst_sh, rst_h.at[c])

    return k(v, src, dst, ex, r, z128)


def _combine_body(p_ref, o_ref):
    o_ref[...] = p_ref[0] + p_ref[1]


def _combine(rstp):
    blk = 1000
    return pl.pallas_call(
        _combine_body,
        grid=(N // blk,),
        in_specs=[pl.BlockSpec((2, blk, HD), lambda i: (0, i, 0))],
        out_specs=pl.BlockSpec((blk, HD), lambda i: (i, 0)),
        out_shape=jax.ShapeDtypeStruct((N, HD), jnp.float32),
    )(rstp)


def kernel(feat, edge_index, W_src, b_src, W_dst, b_dst, W_val, b_val, attn):
    src = edge_index[0]
    dst = edge_index[1]
    w_all_t = jnp.concatenate([W_src, W_dst, W_val], axis=0).T
    b_all = jnp.concatenate([b_src, b_dst, b_val])[None, :]
    el, er, v = _project(feat, w_all_t, b_all)
    attn_b = jnp.broadcast_to(attn.reshape(HD, 1), (HD, 16))
    z4 = jnp.zeros((N * 4,), jnp.float32)
    z128 = jnp.zeros((N, HD), jnp.float32)
    ex, segp = _sc_pass_a(el, er, src, dst, attn_b, z4)
    r = _recip(segp)
    rstp = _sc_pass_d(v, src, dst, ex, r, z128)
    rst = _combine(rstp)
    return rst.reshape(N, HEADS, OUT_FEATS)


# speedup vs baseline: 38.0186x; 4.2739x over previous
import functools

import jax
import jax.numpy as jnp
from jax import lax
from jax.experimental import pallas as pl
from jax.experimental.pallas import tpu as pltpu
from jax.experimental.pallas import tpu_sc as plsc

N = 10000
E = 320000
IN_FEATS = 128
HEADS = 4
OUT_FEATS = 32
HD = HEADS * OUT_FEATS
NEG_SLOPE = 0.2

NW = 32
EPW = E // NW
CC = 80
NCH = EPW // CC
NGR = CC // 16
NPAIR = (NCH - 1) // 2


def _proj_body(feat_ref, w_ref, b_ref, el_ref, er_ref, v_ref):
    r = (
        jnp.dot(feat_ref[...], w_ref[...], preferred_element_type=jnp.float32)
        + b_ref[...]
    )
    el_ref[...] = r[:, :HD]
    er_ref[...] = r[:, HD:2 * HD]
    v_ref[...] = r[:, 2 * HD:]


def _project(feat, w_all_t, b_all):
    blk = 1000
    return pl.pallas_call(
        _proj_body,
        grid=(N // blk,),
        in_specs=[
            pl.BlockSpec((blk, IN_FEATS), lambda i: (i, 0)),
            pl.BlockSpec((IN_FEATS, 3 * HD), lambda i: (0, 0)),
            pl.BlockSpec((1, 3 * HD), lambda i: (0, 0)),
        ],
        out_specs=[
            pl.BlockSpec((blk, HD), lambda i: (i, 0)),
            pl.BlockSpec((blk, HD), lambda i: (i, 0)),
            pl.BlockSpec((blk, HD), lambda i: (i, 0)),
        ],
        out_shape=[
            jax.ShapeDtypeStruct((N, HD), jnp.float32),
            jax.ShapeDtypeStruct((N, HD), jnp.float32),
            jax.ShapeDtypeStruct((N, HD), jnp.float32),
        ],
    )(feat, w_all_t, b_all)


def _sc_pass_a(el, er, src, dst, attn_b, z4):
    mesh = plsc.VectorSubcoreMesh(core_axis_name="c", subcore_axis_name="s")

    @functools.partial(
        pl.kernel,
        mesh=mesh,
        compiler_params=pltpu.CompilerParams(needs_layout_passes=False),
        out_type=[
            jax.ShapeDtypeStruct((E * 4,), jnp.float32),
            jax.ShapeDtypeStruct((2, N * 4), jnp.float32),
        ],
        scratch_types=[
            pltpu.VMEM((CC,), jnp.int32),
            pltpu.VMEM((CC,), jnp.int32),
            pltpu.VMEM((CC * 4,), jnp.int32),
            pltpu.VMEM((CC, HD), jnp.float32),
            pltpu.VMEM((CC, HD), jnp.float32),
            pltpu.VMEM((CC,), jnp.int32),
            pltpu.VMEM((CC,), jnp.int32),
            pltpu.VMEM((CC * 4,), jnp.int32),
            pltpu.VMEM((CC, HD), jnp.float32),
            pltpu.VMEM((CC, HD), jnp.float32),
            pltpu.VMEM((CC * 4,), jnp.float32),
            pltpu.VMEM((HD,), jnp.float32),
            pltpu.VMEM_SHARED((N * 4,), jnp.float32),
            pltpu.SemaphoreType.DMA,
            pltpu.SemaphoreType.DMA,
        ],
    )
    def k(el_h, er_h, src_h, dst_h, attnb_h, z4_h, ex_h, segp_h,
          ids0, idd0, idx40, rel0, rer0,
          ids1, idd1, idx41, rel1, rer1,
          exflat, attn_v, seg_sh, sem0, sem1):
        c = lax.axis_index("c")
        s = lax.axis_index("s")
        w = s * 2 + c
        iota = lax.iota(jnp.int32, 16)
        zeros16 = jnp.zeros((16,), jnp.float32)
        pltpu.sync_copy(attnb_h, attn_v)

        @pl.when(s == 0)
        def _():
            pltpu.sync_copy(z4_h, seg_sh)

        plsc.subcore_barrier()

        bufs = ((ids0, idd0, idx40, rel0, rer0, sem0),
                (ids1, idd1, idx41, rel1, rer1, sem1))

        def issue(kk, b):
            ids, idd, idx4, rel, rer, sem = bufs[b]
            base = w * EPW + kk * CC
            pltpu.sync_copy(src_h.at[pl.ds(base, CC)], ids)
            pltpu.sync_copy(dst_h.at[pl.ds(base, CC)], idd)

            def mkidx(kk2, carry2):
                si = kk2 * 16 + iota
                dstv = plsc.load_gather(idd, [si >> 2])
                idx4[pl.ds(kk2 * 16, 16)] = dstv * 4 + (si & 3)
                return carry2

            lax.fori_loop(0, CC * 4 // 16, mkidx, None)
            pltpu.async_copy(el_h.at[ids], rel, sem)
            pltpu.async_copy(er_h.at[idd], rer, sem)

        av = [attn_v[pl.ds(j * 16, 16)] for j in range(8)]

        def stage(kk, b):
            ids, idd, idx4, rel, rer, sem = bufs[b]
            pltpu.make_async_copy(el_h.at[ids], rel, sem).wait()
            pltpu.make_async_copy(er_h.at[idd], rer, sem).wait()

            def zb(kk2, carry2):
                exflat[pl.ds(kk2 * 16, 16)] = zeros16
                return carry2

            lax.fori_loop(0, CC * 4 // 16, zb, None)

            def edge(e, ecarry):
                t = []
                for j in range(8):
                    sl = pl.ds(j * 16, 16)
                    x = rel[e, sl] + rer[e, sl]
                    x = jnp.maximum(x, x * NEG_SLOPE)
                    t.append(x * av[j])
                lane0 = (e & 3) * 4
                vec = zeros16
                for h in range(HEADS):
                    sh = jnp.sum(t[2 * h] + t[2 * h + 1])
                    vec = vec + jnp.where(iota == lane0 + h, sh, 0.0)
                bl = pl.ds((e >> 2) * 16, 16)
                exflat[bl] = exflat[bl] + vec
                return ecarry

            lax.fori_loop(0, CC, edge, None)

            def expp(kk2, carry2):
                sl = pl.ds(kk2 * 16, 16)
                exflat[sl] = jnp.exp(exflat[sl])
                return carry2

            lax.fori_loop(0, CC * 4 // 16, expp, None)
            base = w * EPW + kk * CC
            pltpu.sync_copy(exflat, ex_h.at[pl.ds(base * 4, CC * 4)])
            pltpu.sync_copy(exflat, seg_sh.at[idx4], add=True)

        issue(0, 0)

        def pair(jj, carry):
            issue(2 * jj + 1, 1)
            stage(2 * jj, 0)
            issue(2 * jj + 2, 0)
            stage(2 * jj + 1, 1)
            return carry

        lax.fori_loop(0, NPAIR, pair, None)
        stage(NCH - 1, 0)
        plsc.subcore_barrier()

        @pl.when(s == 0)
        def _():
            pltpu.sync_copy(seg_sh, segp_h.at[c])

    return k(el, er, src, dst, attn_b, z4)


def _recip_body(p_ref, o_ref):
    o_ref[...] = 1.0 / (p_ref[0] + p_ref[1] + 1e-9)


def _recip(segp):
    return pl.pallas_call(
        _recip_body,
        out_shape=jax.ShapeDtypeStruct((N * 4,), jnp.float32),
    )(segp)


def _sc_pass_d(v, src, dst, ex, r, z128):
    mesh = plsc.VectorSubcoreMesh(core_axis_name="c", subcore_axis_name="s")

    @functools.partial(
        pl.kernel,
        mesh=mesh,
        compiler_params=pltpu.CompilerParams(needs_layout_passes=False),
        out_type=jax.ShapeDtypeStruct((2, N, HD), jnp.float32),
        scratch_types=[
            pltpu.VMEM((CC,), jnp.int32),
            pltpu.VMEM((CC,), jnp.int32),
            pltpu.VMEM((CC * 4,), jnp.int32),
            pltpu.VMEM((CC, HD), jnp.float32),
            pltpu.VMEM((CC * 4,), jnp.float32),
            pltpu.VMEM((CC * 4,), jnp.float32),
            pltpu.VMEM((CC,), jnp.int32),
            pltpu.VMEM((CC,), jnp.int32),
            pltpu.VMEM((CC * 4,), jnp.int32),
            pltpu.VMEM((CC, HD), jnp.float32),
            pltpu.VMEM((CC * 4,), jnp.float32),
            pltpu.VMEM((CC * 4,), jnp.float32),
            pltpu.VMEM((CC, HD), jnp.float32),
            pltpu.VMEM_SHARED((N, HD), jnp.float32),
            pltpu.SemaphoreType.DMA,
            pltpu.SemaphoreType.DMA,
        ],
    )
    def k(v_h, src_h, dst_h, ex_h, r_h, z128_h, rst_h,
          ids0, idd0, idx40, vr0, exf0, rf0,
          ids1, idd1, idx41, vr1, exf1, rf1,
          msg_rows, rst_sh, sem0, sem1):
        c = lax.axis_index("c")
        s = lax.axis_index("s")
        w = s * 2 + c
        iota = lax.iota(jnp.int32, 16)

        @pl.when(s == 0)
        def _():
            pltpu.sync_copy(z128_h, rst_sh)

        plsc.subcore_barrier()

        bufs = ((ids0, idd0, idx40, vr0, exf0, rf0, sem0),
                (ids1, idd1, idx41, vr1, exf1, rf1, sem1))

        def issue(kk, b):
            ids, idd, idx4, vr, exf, rf, sem = bufs[b]
            base = w * EPW + kk * CC
            pltpu.sync_copy(src_h.at[pl.ds(base, CC)], ids)
            pltpu.sync_copy(dst_h.at[pl.ds(base, CC)], idd)

            def mkidx(kk2, carry2):
                si = kk2 * 16 + iota
                dstv = plsc.load_gather(idd, [si >> 2])
                idx4[pl.ds(kk2 * 16, 16)] = dstv * 4 + (si & 3)
                return carry2

            lax.fori_loop(0, CC * 4 // 16, mkidx, None)
            pltpu.async_copy(v_h.at[ids], vr, sem)
            pltpu.async_copy(r_h.at[idx4], rf, sem)
            pltpu.async_copy(ex_h.at[pl.ds(base * 4, CC * 4)], exf, sem)

        def stage(kk, b):
            ids, idd, idx4, vr, exf, rf, sem = bufs[b]
            base = w * EPW + kk * CC
            pltpu.make_async_copy(v_h.at[ids], vr, sem).wait()
            pltpu.make_async_copy(r_h.at[idx4], rf, sem).wait()
            pltpu.make_async_copy(ex_h.at[pl.ds(base * 4, CC * 4)], exf,
                                  sem).wait()

            def amul(kk2, carry2):
                sl = pl.ds(kk2 * 16, 16)
                exf[sl] = exf[sl] * rf[sl]
                return carry2

            lax.fori_loop(0, CC * 4 // 16, amul, None)

            def edge4(e4, ecarry):
                blk = exf[pl.ds(e4 * 16, 16)]
                for sub in range(4):
                    e = e4 * 4 + sub
                    a = [blk[sub * 4 + h] for h in range(HEADS)]
                    for j in range(8):
                        sl = pl.ds(j * 16, 16)
                        msg_rows[e, sl] = vr[e, sl] * a[j // 2]
                return ecarry

            lax.fori_loop(0, CC // 4, edge4, None)
            pltpu.sync_copy(msg_rows, rst_sh.at[idd], add=True)

        issue(0, 0)

        def pair(jj, carry):
            issue(2 * jj + 1, 1)
            stage(2 * jj, 0)
            issue(2 * jj + 2, 0)
            stage(2 * jj + 1, 1)
            return carry

        lax.fori_loop(0, NPAIR, pair, None)
        stage(NCH - 1, 0)
        plsc.subcore_barrier()

        @pl.when(s == 0)
        def _():
            pltpu.sync_copy(rst_sh, rst_h.at[c])

    return k(v, src, dst, ex, r, z128)


def _combine_body(p_ref, o_ref):
    o_ref[...] = p_ref[0] + p_ref[1]


def _combine(rstp):
    blk = 1000
    return pl.pallas_call(
        _combine_body,
        grid=(N // blk,),
        in_specs=[pl.BlockSpec((2, blk, HD), lambda i: (0, i, 0))],
        out_specs=pl.BlockSpec((blk, HD), lambda i: (i, 0)),
        out_shape=jax.ShapeDtypeStruct((N, HD), jnp.float32),
    )(rstp)


def kernel(feat, edge_index, W_src, b_src, W_dst, b_dst, W_val, b_val, attn):
    src = edge_index[0]
    dst = edge_index[1]
    w_all_t = jnp.concatenate([W_src, W_dst, W_val], axis=0).T
    b_all = jnp.concatenate([b_src, b_dst, b_val])[None, :]
    el, er, v = _project(feat, w_all_t, b_all)
    attn_b = attn.reshape(HD)
    z4 = jnp.zeros((N * 4,), jnp.float32)
    z128 = jnp.zeros((N, HD), jnp.float32)
    ex, segp = _sc_pass_a(el, er, src, dst, attn_b, z4)
    r = _recip(segp)
    rstp = _sc_pass_d(v, src, dst, ex, r, z128)
    rst = _combine(rstp)
    return rst.reshape(N, HEADS, OUT_FEATS)
